# baseline (speedup 1.0000x reference)
import jax
import jax.numpy as jnp
from jax import lax
from jax.experimental import pallas as pl
from jax.experimental.pallas import tpu as pltpu
from jax.experimental.pallas import tpu_sc as plsc

N = 10000
E = 320000
H = 128
HH = H // 2

NC, NS = 2, 16
CH = 128
NCHUNK = 164
EPT = NCHUNK * CH
EPAD = NS * EPT
NPAD = 10240
RPT = NPAD // NS
CNTW = 8
NBUF = 4

BLK = 1024
GRID = NPAD // BLK

_f32 = jnp.float32
_bf16 = jnp.bfloat16


def _seg_body(src_hbm, dst_hbm, h_hbm, ones_hbm, zrow_hbm, zcnt_hbm,
              flag_hbm, acc_hbm, cnt_hbm,
              src_v, dst_v, rows_v, ones_v, zbuf, cbuf, flag_sm,
              acc_sh, cnt_sh, gsem, ssem):
    cid = lax.axis_index("c")
    sid = lax.axis_index("s")
    pltpu.sync_copy(flag_hbm, flag_sm)
    do_cnt = jnp.max(flag_sm[...]) == 1

    pltpu.sync_copy(zrow_hbm, zbuf)
    row0 = sid * RPT
    for k in range(RPT // 128):
        pltpu.async_copy(zbuf, acc_sh.at[pl.ds(row0 + k * 128, 128)], ssem)

    @pl.when(do_cnt)
    def _():
        pltpu.sync_copy(ones_hbm, ones_v)
        pltpu.sync_copy(zcnt_hbm, cbuf)
        pltpu.sync_copy(cbuf, cnt_sh.at[pl.ds(row0, RPT)])

    pltpu.async_copy(src_hbm.at[pl.ds(sid * NCHUNK, NCHUNK)], src_v, gsem)
    pltpu.async_copy(dst_hbm.at[pl.ds(sid * NCHUNK, NCHUNK)], dst_v, gsem)
    for k in range(RPT // 128):
        pltpu.make_async_copy(zbuf, acc_sh.at[pl.ds(row0 + k * 128, 128)],
                              ssem).wait()
    pltpu.make_async_copy(src_hbm.at[pl.ds(sid * NCHUNK, NCHUNK)], src_v,
                          gsem).wait()
    pltpu.make_async_copy(dst_hbm.at[pl.ds(sid * NCHUNK, NCHUNK)], dst_v,
                          gsem).wait()
    plsc.subcore_barrier()

    htab = h_hbm.at[cid]

    for pb in range(2):
        pltpu.async_copy(htab.at[src_v.at[pb]], rows_v.at[pb], gsem)

    def gbody(g, carry):
        for b in range(NBUF):
            j = g * NBUF + b
            pltpu.make_async_copy(htab.at[src_v.at[j]], rows_v.at[b],
                                  gsem).wait()
            pltpu.async_copy(rows_v.at[b], acc_sh.at[dst_v.at[j]], ssem,
                             add=True)

            @pl.when(j >= 1)
            def _():
                pltpu.make_async_copy(rows_v.at[(b + NBUF - 1) % NBUF],
                                      acc_sh.at[dst_v.at[j - 1]],
                                      ssem).wait()

            @pl.when(j + 2 < NCHUNK)
            def _():
                pltpu.async_copy(htab.at[src_v.at[j + 2]],
                                 rows_v.at[(b + 2) % NBUF], gsem)

            @pl.when(do_cnt & ((j < NCHUNK // 2) == (cid == 0)))
            def _():
                pltpu.sync_copy(ones_v, cnt_sh.at[dst_v.at[j]], add=True)
        return carry

    lax.fori_loop(0, NCHUNK // NBUF, gbody, 0)
    pltpu.make_async_copy(rows_v.at[(NCHUNK - 1) % NBUF],
                          acc_sh.at[dst_v.at[NCHUNK - 1]], ssem).wait()
    plsc.subcore_barrier()

    for k in range(RPT // 128):
        b = k % NBUF
        r = row0 + k * 128
        if k >= NBUF:
            rp = row0 + (k - NBUF) * 128
            pltpu.make_async_copy(rows_v.at[b],
                                  acc_hbm.at[cid].at[pl.ds(rp, 128)],
                                  ssem).wait()
        pltpu.sync_copy(acc_sh.at[pl.ds(r, 128)], rows_v.at[b])
        pltpu.async_copy(rows_v.at[b], acc_hbm.at[cid].at[pl.ds(r, 128)],
                         ssem)
    for k in range(RPT // 128 - NBUF, RPT // 128):
        r = row0 + k * 128
        pltpu.make_async_copy(rows_v.at[k % NBUF],
                              acc_hbm.at[cid].at[pl.ds(r, 128)], ssem).wait()

    @pl.when(do_cnt)
    def _():
        pltpu.sync_copy(cnt_sh.at[pl.ds(row0, RPT)], cbuf)
        pltpu.sync_copy(cbuf, cnt_hbm.at[cid].at[pl.ds(row0, RPT)])


_seg_kernel = pl.kernel(
    _seg_body,
    out_type=(
        pltpu.HBM((NC, NPAD, HH), _bf16),
        pltpu.HBM((NC, NPAD, CNTW), _f32),
    ),
    mesh=plsc.VectorSubcoreMesh(core_axis_name="c", subcore_axis_name="s"),
    compiler_params=pltpu.CompilerParams(use_tc_tiling_on_sc=False,
                                         needs_layout_passes=False),
    scratch_types=[
        pltpu.VMEM((NCHUNK, CH), jnp.int32),
        pltpu.VMEM((NCHUNK, CH), jnp.int32),
        pltpu.VMEM((NBUF, CH, HH), _bf16),
        pltpu.VMEM((CH, CNTW), _f32),
        pltpu.VMEM((128, HH), _bf16),
        pltpu.VMEM((RPT, CNTW), _f32),
        pltpu.VMEM((16,), jnp.int32),
        pltpu.VMEM_SHARED((NPAD, HH), _bf16),
        pltpu.VMEM_SHARED((NPAD, CNTW), _f32),
        pltpu.SemaphoreType.DMA,
        pltpu.SemaphoreType.DMA,
    ],
)


def _mm_h_body(x_ref, W_ref, h_ref):
    hfull = jnp.dot(x_ref[...], W_ref[...], preferred_element_type=_f32)
    h_ref[0] = hfull[:, :HH].astype(_bf16)
    h_ref[1] = hfull[:, HH:].astype(_bf16)


def _mm_xc_body(x_ref, lW_ref, lb_ref, xc_ref):
    xc_ref[...] = lax.dot_general(
        x_ref[...], lW_ref[...], (((1,), (1,)), ((), ())),
        preferred_element_type=_f32) + lb_ref[...]


def _combine(p_ref, c_ref, h_ref, xc_ref):
    s = jnp.concatenate(
        [p_ref[0].astype(_f32) + h_ref[0].astype(_f32),
         p_ref[1].astype(_f32) + h_ref[1].astype(_f32)], axis=-1)
    cnt = c_ref[0, :, 0:1] + c_ref[1, :, 0:1] + 1.0
    return jnp.maximum(s / jnp.maximum(cnt, 1.0) + xc_ref[...], 0.0)


def _comb_body(p_ref, c_ref, h_ref, xc_ref, W_ref, h2_ref, x1_ref):
    x1 = _combine(p_ref, c_ref, h_ref, xc_ref)
    x1_ref[...] = x1
    hfull = jnp.dot(x1, W_ref[...], preferred_element_type=_f32)
    h2_ref[0] = hfull[:, :HH].astype(_bf16)
    h2_ref[1] = hfull[:, HH:].astype(_bf16)


def _final_body(p_ref, c_ref, h_ref, xc_ref, W3_ref, b3_ref, out_ref):
    x2 = _combine(p_ref, c_ref, h_ref, xc_ref)
    logits = lax.dot_general(
        x2, W3_ref[...], (((1,), (1,)), ((), ())),
        preferred_element_type=_f32) + b3_ref[...]
    m = jnp.max(logits, axis=-1, keepdims=True)
    lse = jnp.log(jnp.sum(jnp.exp(logits - m), axis=-1, keepdims=True)) + m
    out_ref[...] = logits - lse


def _row_spec(width=H):
    return pl.BlockSpec((BLK, width), lambda i: (i, 0))


def _split_spec(width=HH):
    return pl.BlockSpec((NC, BLK, width), lambda i: (0, i, 0))


_w_spec = pl.BlockSpec((H, H), lambda i: (0, 0))
_b_spec = pl.BlockSpec((1, H), lambda i: (0, 0))

_split_shape = jax.ShapeDtypeStruct((NC, NPAD, HH), _bf16)
_full_shape = jax.ShapeDtypeStruct((NPAD, H), _f32)

_x_spec = pl.BlockSpec((BLK, H), lambda i: (i, 0))

_mm_h = pl.pallas_call(
    _mm_h_body,
    grid=(GRID,),
    in_specs=[_x_spec, _w_spec],
    out_specs=_split_spec(),
    out_shape=_split_shape,
)

_mm_xc = pl.pallas_call(
    _mm_xc_body,
    grid=(GRID,),
    in_specs=[_x_spec, _w_spec, _b_spec],
    out_specs=_row_spec(),
    out_shape=_full_shape,
)

_comb = pl.pallas_call(
    _comb_body,
    grid=(GRID,),
    in_specs=[_split_spec(), _split_spec(CNTW), _split_spec(), _row_spec(),
              _w_spec],
    out_specs=[_split_spec(), _row_spec()],
    out_shape=[_split_shape, _full_shape],
)

_final = pl.pallas_call(
    _final_body,
    grid=(GRID,),
    in_specs=[_split_spec(), _split_spec(CNTW), _split_spec(), _row_spec(),
              _w_spec, _b_spec],
    out_specs=_row_spec(),
    out_shape=jax.ShapeDtypeStruct((NPAD, H), _f32),
)


def kernel(x0, edge_index, W1, linW1, linb1, W2, linW2, linb2, W3, b3):
    pad_e = EPAD - E
    srcp = jnp.concatenate(
        [edge_index[0], jnp.zeros((pad_e,), jnp.int32)]).reshape(-1, CH)
    dstp = jnp.concatenate(
        [edge_index[1],
         N + jnp.arange(pad_e, dtype=jnp.int32) % (NPAD - N)]).reshape(-1, CH)
    ones = jnp.ones((CH, CNTW), _f32)
    zrow = jnp.zeros((128, HH), _bf16)
    zcnt = jnp.zeros((RPT, CNTW), _f32)
    lb1 = linb1.reshape(1, H)
    lb2 = linb2.reshape(1, H)
    b3r = b3.reshape(1, H)

    f1 = jnp.ones((16,), jnp.int32)
    f0 = jnp.zeros((16,), jnp.int32)

    x0p = jnp.pad(x0, ((0, NPAD - N), (0, 0)))
    h1 = _mm_h(x0p, W1)
    p1, c1 = _seg_kernel(srcp, dstp, h1, ones, zrow, zcnt, f1)
    xc1 = _mm_xc(x0p, linW1, lb1)
    h2, x1 = _comb(p1, c1, h1, xc1, W2)
    p2, _ = _seg_kernel(srcp, dstp, h2, ones, zrow, zcnt, f0)
    xc2 = _mm_xc(x1, linW2, lb2)
    return _final(p2, c1, h2, xc2, W3, b3r)[:N]


# speedup vs baseline: 1.2081x; 1.2081x over previous
import jax
import jax.numpy as jnp
from jax import lax
from jax.experimental import pallas as pl
from jax.experimental.pallas import tpu as pltpu
from jax.experimental.pallas import tpu_sc as plsc

N = 10000
E = 320000
H = 128
HH = H // 2

NC, NS = 2, 16
CH = 128
NCHUNK = 162
EPT = NCHUNK * CH
EPAD = NS * EPT
NPAD = 10240
RPT = NPAD // NS
CNTW = 8
NBUF = 3

BLK = 1024
GRID = NPAD // BLK

_f32 = jnp.float32
_bf16 = jnp.bfloat16


def _seg_body(src_hbm, dst_hbm, h_hbm, ones_hbm, zrow_hbm, zcnt_hbm,
              flag_hbm, acc_hbm, cnt_hbm,
              src_v, dst_v, rows_v, ones_v, zbuf, cbuf, flag_sm,
              acc_sh, cnt_sh, gsem, ssem):
    cid = lax.axis_index("c")
    sid = lax.axis_index("s")
    pltpu.sync_copy(flag_hbm, flag_sm)
    do_cnt = jnp.max(flag_sm[...]) == 1

    pltpu.sync_copy(zrow_hbm, zbuf)
    row0 = sid * RPT
    for k in range(RPT // 128):
        pltpu.async_copy(zbuf, acc_sh.at[pl.ds(row0 + k * 128, 128)], ssem)

    @pl.when(do_cnt)
    def _():
        pltpu.sync_copy(ones_hbm, ones_v)
        pltpu.sync_copy(zcnt_hbm, cbuf)
        pltpu.sync_copy(cbuf, cnt_sh.at[pl.ds(row0, RPT)])

    pltpu.async_copy(src_hbm.at[pl.ds(sid * NCHUNK, NCHUNK)], src_v, gsem)
    pltpu.async_copy(dst_hbm.at[pl.ds(sid * NCHUNK, NCHUNK)], dst_v, gsem)
    for k in range(RPT // 128):
        pltpu.make_async_copy(zbuf, acc_sh.at[pl.ds(row0 + k * 128, 128)],
                              ssem).wait()
    pltpu.make_async_copy(src_hbm.at[pl.ds(sid * NCHUNK, NCHUNK)], src_v,
                          gsem).wait()
    pltpu.make_async_copy(dst_hbm.at[pl.ds(sid * NCHUNK, NCHUNK)], dst_v,
                          gsem).wait()
    plsc.subcore_barrier()

    htab = h_hbm.at[cid]

    for pb in range(NBUF - 1):
        pltpu.async_copy(htab.at[src_v.at[pb]], rows_v.at[pb], gsem)

    def gbody(g, carry):
        for b in range(NBUF):
            j = g * NBUF + b
            pltpu.make_async_copy(htab.at[src_v.at[j]], rows_v.at[b],
                                  gsem).wait()

            @pl.when(j + NBUF - 1 < NCHUNK)
            def _():
                pltpu.async_copy(htab.at[src_v.at[j + NBUF - 1]],
                                 rows_v.at[(b + NBUF - 1) % NBUF], gsem)

            pltpu.sync_copy(rows_v.at[b], acc_sh.at[dst_v.at[j]], add=True)

            @pl.when(do_cnt & ((j < NCHUNK // 2) == (cid == 0)))
            def _():
                pltpu.sync_copy(ones_v, cnt_sh.at[dst_v.at[j]], add=True)
        return carry

    lax.fori_loop(0, NCHUNK // NBUF, gbody, 0)
    plsc.subcore_barrier()

    for k in range(RPT // 128):
        b = k % NBUF
        r = row0 + k * 128
        if k >= NBUF:
            rp = row0 + (k - NBUF) * 128
            pltpu.make_async_copy(rows_v.at[b],
                                  acc_hbm.at[cid].at[pl.ds(rp, 128)],
                                  ssem).wait()
        pltpu.sync_copy(acc_sh.at[pl.ds(r, 128)], rows_v.at[b])
        pltpu.async_copy(rows_v.at[b], acc_hbm.at[cid].at[pl.ds(r, 128)],
                         ssem)
    for k in range(RPT // 128 - NBUF, RPT // 128):
        r = row0 + k * 128
        pltpu.make_async_copy(rows_v.at[k % NBUF],
                              acc_hbm.at[cid].at[pl.ds(r, 128)], ssem).wait()

    @pl.when(do_cnt)
    def _():
        pltpu.sync_copy(cnt_sh.at[pl.ds(row0, RPT)], cbuf)
        pltpu.sync_copy(cbuf, cnt_hbm.at[cid].at[pl.ds(row0, RPT)])


_seg_kernel = pl.kernel(
    _seg_body,
    out_type=(
        pltpu.HBM((NC, NPAD, HH), _bf16),
        pltpu.HBM((NC, NPAD, CNTW), _f32),
    ),
    mesh=plsc.VectorSubcoreMesh(core_axis_name="c", subcore_axis_name="s"),
    compiler_params=pltpu.CompilerParams(use_tc_tiling_on_sc=False,
                                         needs_layout_passes=False),
    scratch_types=[
        pltpu.VMEM((NCHUNK, CH), jnp.int32),
        pltpu.VMEM((NCHUNK, CH), jnp.int32),
        pltpu.VMEM((NBUF, CH, HH), _bf16),
        pltpu.VMEM((CH, CNTW), _f32),
        pltpu.VMEM((128, HH), _bf16),
        pltpu.VMEM((RPT, CNTW), _f32),
        pltpu.VMEM((16,), jnp.int32),
        pltpu.VMEM_SHARED((NPAD, HH), _bf16),
        pltpu.VMEM_SHARED((NPAD, CNTW), _f32),
        pltpu.SemaphoreType.DMA,
        pltpu.SemaphoreType.DMA,
    ],
)


def _mm_h_body(x_ref, W_ref, h_ref):
    hfull = jnp.dot(x_ref[...], W_ref[...], preferred_element_type=_f32)
    h_ref[0] = hfull[:, :HH].astype(_bf16)
    h_ref[1] = hfull[:, HH:].astype(_bf16)


def _mm_xc_body(x_ref, lW_ref, lb_ref, xc_ref):
    xc_ref[...] = lax.dot_general(
        x_ref[...], lW_ref[...], (((1,), (1,)), ((), ())),
        preferred_element_type=_f32) + lb_ref[...]


def _combine(p_ref, c_ref, h_ref, xc_ref):
    s = jnp.concatenate(
        [p_ref[0].astype(_f32) + h_ref[0].astype(_f32),
         p_ref[1].astype(_f32) + h_ref[1].astype(_f32)], axis=-1)
    cnt = c_ref[0, :, 0:1] + c_ref[1, :, 0:1] + 1.0
    return jnp.maximum(s / jnp.maximum(cnt, 1.0) + xc_ref[...], 0.0)


def _comb_body(p_ref, c_ref, h_ref, xc_ref, W_ref, h2_ref, x1_ref):
    x1 = _combine(p_ref, c_ref, h_ref, xc_ref)
    x1_ref[...] = x1
    hfull = jnp.dot(x1, W_ref[...], preferred_element_type=_f32)
    h2_ref[0] = hfull[:, :HH].astype(_bf16)
    h2_ref[1] = hfull[:, HH:].astype(_bf16)


def _final_body(p_ref, c_ref, h_ref, xc_ref, W3_ref, b3_ref, out_ref):
    x2 = _combine(p_ref, c_ref, h_ref, xc_ref)
    logits = lax.dot_general(
        x2, W3_ref[...], (((1,), (1,)), ((), ())),
        preferred_element_type=_f32) + b3_ref[...]
    m = jnp.max(logits, axis=-1, keepdims=True)
    lse = jnp.log(jnp.sum(jnp.exp(logits - m), axis=-1, keepdims=True)) + m
    out_ref[...] = logits - lse


def _row_spec(width=H):
    return pl.BlockSpec((BLK, width), lambda i: (i, 0))


def _split_spec(width=HH):
    return pl.BlockSpec((NC, BLK, width), lambda i: (0, i, 0))


_w_spec = pl.BlockSpec((H, H), lambda i: (0, 0))
_b_spec = pl.BlockSpec((1, H), lambda i: (0, 0))

_split_shape = jax.ShapeDtypeStruct((NC, NPAD, HH), _bf16)
_full_shape = jax.ShapeDtypeStruct((NPAD, H), _f32)

_x_spec = pl.BlockSpec((BLK, H), lambda i: (i, 0))

_mm_h = pl.pallas_call(
    _mm_h_body,
    grid=(GRID,),
    in_specs=[_x_spec, _w_spec],
    out_specs=_split_spec(),
    out_shape=_split_shape,
)

_mm_xc = pl.pallas_call(
    _mm_xc_body,
    grid=(GRID,),
    in_specs=[_x_spec, _w_spec, _b_spec],
    out_specs=_row_spec(),
    out_shape=_full_shape,
)

_comb = pl.pallas_call(
    _comb_body,
    grid=(GRID,),
    in_specs=[_split_spec(), _split_spec(CNTW), _split_spec(), _row_spec(),
              _w_spec],
    out_specs=[_split_spec(), _row_spec()],
    out_shape=[_split_shape, _full_shape],
)

_final = pl.pallas_call(
    _final_body,
    grid=(GRID,),
    in_specs=[_split_spec(), _split_spec(CNTW), _split_spec(), _row_spec(),
              _w_spec, _b_spec],
    out_specs=_row_spec(),
    out_shape=jax.ShapeDtypeStruct((NPAD, H), _f32),
)


def kernel(x0, edge_index, W1, linW1, linb1, W2, linW2, linb2, W3, b3):
    pad_e = EPAD - E
    srcp = jnp.concatenate(
        [edge_index[0], jnp.zeros((pad_e,), jnp.int32)]).reshape(-1, CH)
    dstp = jnp.concatenate(
        [edge_index[1],
         N + jnp.arange(pad_e, dtype=jnp.int32) % (NPAD - N)]).reshape(-1, CH)
    ones = jnp.ones((CH, CNTW), _f32)
    zrow = jnp.zeros((128, HH), _bf16)
    zcnt = jnp.zeros((RPT, CNTW), _f32)
    lb1 = linb1.reshape(1, H)
    lb2 = linb2.reshape(1, H)
    b3r = b3.reshape(1, H)

    f1 = jnp.ones((16,), jnp.int32)
    f0 = jnp.zeros((16,), jnp.int32)

    x0p = jnp.pad(x0, ((0, NPAD - N), (0, 0)))
    h1 = _mm_h(x0p, W1)
    p1, c1 = _seg_kernel(srcp, dstp, h1, ones, zrow, zcnt, f1)
    xc1 = _mm_xc(x0p, linW1, lb1)
    h2, x1 = _comb(p1, c1, h1, xc1, W2)
    p2, _ = _seg_kernel(srcp, dstp, h2, ones, zrow, zcnt, f0)
    xc2 = _mm_xc(x1, linW2, lb2)
    return _final(p2, c1, h2, xc2, W3, b3r)[:N]
